# baseline (speedup 1.0000x reference)
import functools
import numpy as np
import jax
import jax.numpy as jnp
from jax import lax
from jax.experimental import pallas as pl
from jax.experimental.pallas import tpu as pltpu
from jax.experimental.pallas import tpu_sc as plsc

VOCAB_SIZE = 1000000
EMBED_DIM = 32
BATCH = 4096
SEQ_LEN = 200
MAX_WAVELENGTH = 10000.0

ROWS = BATCH * SEQ_LEN
NC, NS = 2, 16
NW = NC * NS
ROWS_PER_W = ROWS // NW
SUB = 100
NSUB = 8
CHUNK = SUB * NSUB
NCHUNK = ROWS_PER_W // CHUNK
NT = NCHUNK // 2
TMPL_ROWS = 5 * SEQ_LEN


def _pos_encoding_np():
    positions = np.arange(SEQ_LEN, dtype=np.float32)
    idx = np.arange(EMBED_DIM)
    min_freq = 1.0 / MAX_WAVELENGTH
    timescales = np.power(
        min_freq, (2.0 * (idx // 2).astype(np.float32)) / float(EMBED_DIM)
    ).astype(np.float32)
    angles = positions[:, None] * timescales[None, :]
    enc = np.where((idx % 2) == 0, np.sin(angles), np.cos(angles))
    return enc.astype(np.float32)


_POS_TMPL = np.tile(_pos_encoding_np(), (TMPL_ROWS // SEQ_LEN, 1)).astype(np.float32)


def _body(table, idx_hbm, pos_hbm, out, idx_v, rows_v, tmpl_v, sem_in, sem_g, sem_out):
    cid = lax.axis_index("c")
    sid = lax.axis_index("s")
    wid = sid * NC + cid

    @pl.when(sid == 0)
    def _():
        pltpu.sync_copy(pos_hbm, tmpl_v)

    plsc.subcore_barrier()

    pltpu.sync_copy(idx_hbm.at[wid], idx_v)

    def pre(c, b):
        p = lax.rem(c * CHUNK, SEQ_LEN)
        pltpu.async_copy(tmpl_v.at[pl.ds(p, CHUNK)], rows_v.at[b], sem_in.at[b])

    def wait_pre(c, b):
        p = lax.rem(c * CHUNK, SEQ_LEN)
        pltpu.make_async_copy(
            tmpl_v.at[pl.ds(p, CHUNK)], rows_v.at[b], sem_in.at[b]
        ).wait()

    def fire_gathers(c, b):
        for k in range(NSUB):
            pltpu.async_copy(
                table.at[idx_v.at[c, k]],
                rows_v.at[b, pl.ds(k * SUB, SUB)],
                sem_g.at[b],
                add=True,
            )

    def wait_gathers(c, b):
        for k in range(NSUB):
            pltpu.make_async_copy(
                table.at[idx_v.at[c, k]],
                rows_v.at[b, pl.ds(k * SUB, SUB)],
                sem_g.at[b],
            ).wait()

    def start_out(c, b):
        pltpu.async_copy(
            rows_v.at[b],
            out.at[pl.ds((wid * NCHUNK + c) * CHUNK, CHUNK)],
            sem_out.at[b],
        )

    def wait_out(c, b):
        pltpu.make_async_copy(
            rows_v.at[b],
            out.at[pl.ds((wid * NCHUNK + c) * CHUNK, CHUNK)],
            sem_out.at[b],
        ).wait()

    last = NCHUNK - 1

    pre(0, 0)
    wait_pre(0, 0)
    fire_gathers(0, 0)
    pre(1, 1)

    def pair_body(t, carry):
        c0 = 2 * t
        c1 = 2 * t + 1
        n0 = jnp.minimum(c0 + 2, last)
        n1 = jnp.minimum(c1 + 2, last)
        wait_gathers(c0, 0)
        start_out(c0, 0)
        wait_pre(c1, 1)
        fire_gathers(c1, 1)
        wait_out(c0, 0)
        pre(n0, 0)
        wait_gathers(c1, 1)
        start_out(c1, 1)
        wait_pre(n0, 0)
        fire_gathers(n0, 0)
        wait_out(c1, 1)
        pre(n1, 1)
        return carry

    lax.fori_loop(0, NT, pair_body, 0)

    wait_gathers(last, 0)
    wait_pre(last, 1)


@functools.partial(jax.jit, donate_argnums=())
def _emb(table, idx4, pos_tmpl):
    mesh = plsc.VectorSubcoreMesh(core_axis_name="c", subcore_axis_name="s")
    run = pl.kernel(
        _body,
        mesh=mesh,
        compiler_params=pltpu.CompilerParams(use_tc_tiling_on_sc=False),
        out_type=jax.ShapeDtypeStruct((ROWS, EMBED_DIM), jnp.float32),
        scratch_types=[
            pltpu.VMEM((NCHUNK, NSUB, SUB), jnp.int32),
            pltpu.VMEM((2, CHUNK, EMBED_DIM), jnp.float32),
            pltpu.VMEM_SHARED((TMPL_ROWS, EMBED_DIM), jnp.float32),
            pltpu.SemaphoreType.DMA((2,)),
            pltpu.SemaphoreType.DMA((2,)),
            pltpu.SemaphoreType.DMA((2,)),
        ],
    )
    return run(table, idx4, pos_tmpl)


def kernel(token_emb, x):
    idx4 = x.astype(jnp.int32).reshape(NW, NCHUNK, NSUB, SUB)
    out = _emb(token_emb, idx4, _POS_TMPL)
    return out.reshape(BATCH, SEQ_LEN, EMBED_DIM)


# speedup vs baseline: 1.0026x; 1.0026x over previous
import functools
import numpy as np
import jax
import jax.numpy as jnp
from jax import lax
from jax.experimental import pallas as pl
from jax.experimental.pallas import tpu as pltpu
from jax.experimental.pallas import tpu_sc as plsc

VOCAB_SIZE = 1000000
EMBED_DIM = 32
BATCH = 4096
SEQ_LEN = 200
MAX_WAVELENGTH = 10000.0

ROWS = BATCH * SEQ_LEN
NC, NS = 2, 16
NW = NC * NS
ROWS_PER_W = ROWS // NW
SUB = 100
NSUB = 8
CHUNK = SUB * NSUB
NCHUNK = ROWS_PER_W // CHUNK
NT = NCHUNK // 2


def _pos_encoding_np():
    positions = np.arange(SEQ_LEN, dtype=np.float32)
    idx = np.arange(EMBED_DIM)
    min_freq = 1.0 / MAX_WAVELENGTH
    timescales = np.power(
        min_freq, (2.0 * (idx // 2).astype(np.float32)) / float(EMBED_DIM)
    ).astype(np.float32)
    angles = positions[:, None] * timescales[None, :]
    enc = np.where((idx % 2) == 0, np.sin(angles), np.cos(angles))
    return enc.astype(np.float32)


_POS_TMPL = np.tile(_pos_encoding_np(), (CHUNK // SEQ_LEN, 1)).astype(np.float32)


def _body(table, idx_hbm, pos_hbm, out, idx_v, rows_v, tmpl_v, sem_in, sem_g, sem_out):
    cid = lax.axis_index("c")
    sid = lax.axis_index("s")
    wid = sid * NC + cid

    @pl.when(sid == 0)
    def _():
        pltpu.sync_copy(pos_hbm, tmpl_v)

    plsc.subcore_barrier()

    pltpu.sync_copy(idx_hbm.at[wid], idx_v)

    def pre(b):
        pltpu.async_copy(tmpl_v, rows_v.at[b], sem_in.at[b])

    def wait_pre(b):
        pltpu.make_async_copy(tmpl_v, rows_v.at[b], sem_in.at[b]).wait()

    def fire_gathers(c, b):
        for k in range(NSUB):
            pltpu.async_copy(
                table.at[idx_v.at[c, k]],
                rows_v.at[b, pl.ds(k * SUB, SUB)],
                sem_g.at[b],
                add=True,
            )

    def wait_gathers(c, b):
        for k in range(NSUB):
            pltpu.make_async_copy(
                table.at[idx_v.at[c, k]],
                rows_v.at[b, pl.ds(k * SUB, SUB)],
                sem_g.at[b],
            ).wait()

    def start_out(c, b):
        pltpu.async_copy(
            rows_v.at[b],
            out.at[pl.ds((wid * NCHUNK + c) * CHUNK, CHUNK)],
            sem_out.at[b],
        )

    def wait_out(c, b):
        pltpu.make_async_copy(
            rows_v.at[b],
            out.at[pl.ds((wid * NCHUNK + c) * CHUNK, CHUNK)],
            sem_out.at[b],
        ).wait()

    pre(0)
    wait_pre(0)
    fire_gathers(0, 0)
    pre(1)

    def pair_body(t, carry):
        c0 = 2 * t
        c1 = 2 * t + 1
        wait_gathers(c0, 0)
        start_out(c0, 0)
        wait_pre(1)
        fire_gathers(c1, 1)
        wait_out(c0, 0)
        pre(0)
        wait_gathers(c1, 1)
        start_out(c1, 1)
        wait_pre(0)
        fire_gathers(c0 + 2, 0)
        wait_out(c1, 1)
        pre(1)
        return carry

    lax.fori_loop(0, NT - 1, pair_body, 0)

    c0 = NCHUNK - 2
    c1 = NCHUNK - 1
    wait_gathers(c0, 0)
    start_out(c0, 0)
    wait_pre(1)
    fire_gathers(c1, 1)
    wait_out(c0, 0)
    wait_gathers(c1, 1)
    start_out(c1, 1)
    wait_out(c1, 1)


@functools.partial(jax.jit, donate_argnums=())
def _emb(table, idx4, pos_tmpl):
    mesh = plsc.VectorSubcoreMesh(core_axis_name="c", subcore_axis_name="s")
    run = pl.kernel(
        _body,
        mesh=mesh,
        compiler_params=pltpu.CompilerParams(use_tc_tiling_on_sc=False),
        out_type=jax.ShapeDtypeStruct((ROWS, EMBED_DIM), jnp.float32),
        scratch_types=[
            pltpu.VMEM((NCHUNK, NSUB, SUB), jnp.int32),
            pltpu.VMEM((2, CHUNK, EMBED_DIM), jnp.float32),
            pltpu.VMEM_SHARED((CHUNK, EMBED_DIM), jnp.float32),
            pltpu.SemaphoreType.DMA((2,)),
            pltpu.SemaphoreType.DMA((2,)),
            pltpu.SemaphoreType.DMA((2,)),
        ],
    )
    return run(table, idx4, pos_tmpl)


def kernel(token_emb, x):
    idx4 = x.astype(jnp.int32).reshape(NW, NCHUNK, NSUB, SUB)
    out = _emb(token_emb, idx4, _POS_TMPL)
    return out.reshape(BATCH, SEQ_LEN, EMBED_DIM)


# speedup vs baseline: 1.0033x; 1.0007x over previous
import functools
import numpy as np
import jax
import jax.numpy as jnp
from jax import lax
from jax.experimental import pallas as pl
from jax.experimental.pallas import tpu as pltpu
from jax.experimental.pallas import tpu_sc as plsc

VOCAB_SIZE = 1000000
EMBED_DIM = 32
BATCH = 4096
SEQ_LEN = 200
MAX_WAVELENGTH = 10000.0

ROWS = BATCH * SEQ_LEN
NC, NS = 2, 16
NW = NC * NS
ROWS_PER_W = ROWS // NW
SUB = 100
NSUB = 8
CHUNK = SUB * NSUB
NCHUNK = ROWS_PER_W // CHUNK
NT = NCHUNK // 2


def _pos_encoding_np():
    positions = np.arange(SEQ_LEN, dtype=np.float32)
    idx = np.arange(EMBED_DIM)
    min_freq = 1.0 / MAX_WAVELENGTH
    timescales = np.power(
        min_freq, (2.0 * (idx // 2).astype(np.float32)) / float(EMBED_DIM)
    ).astype(np.float32)
    angles = positions[:, None] * timescales[None, :]
    enc = np.where((idx % 2) == 0, np.sin(angles), np.cos(angles))
    return enc.astype(np.float32)


_POS_TMPL = np.tile(_pos_encoding_np(), (CHUNK // SEQ_LEN, 1)).astype(np.float32)


def _body(
    table, idx_hbm, pos_hbm, out, idx_v, rows_v, tmpl_v, sem_in, sem_g, sem_out, sem_idx
):
    cid = lax.axis_index("c")
    sid = lax.axis_index("s")
    wid = sid * NC + cid

    @pl.when(sid == 0)
    def _():
        pltpu.sync_copy(pos_hbm, tmpl_v)

    plsc.subcore_barrier()

    pltpu.sync_copy(idx_hbm.at[wid, pl.ds(0, 2)], idx_v.at[pl.ds(0, 2)])
    pltpu.async_copy(
        idx_hbm.at[wid, pl.ds(2, NCHUNK - 2)], idx_v.at[pl.ds(2, NCHUNK - 2)], sem_idx
    )

    def pre(b):
        pltpu.async_copy(tmpl_v, rows_v.at[b], sem_in.at[b])

    def wait_pre(b):
        pltpu.make_async_copy(tmpl_v, rows_v.at[b], sem_in.at[b]).wait()

    def fire_gathers(c, b):
        for k in range(NSUB):
            pltpu.async_copy(
                table.at[idx_v.at[c, k]],
                rows_v.at[b, pl.ds(k * SUB, SUB)],
                sem_g.at[b],
                add=True,
            )

    def wait_gathers(c, b):
        for k in range(NSUB):
            pltpu.make_async_copy(
                table.at[idx_v.at[c, k]],
                rows_v.at[b, pl.ds(k * SUB, SUB)],
                sem_g.at[b],
            ).wait()

    def start_out(c, b):
        pltpu.async_copy(
            rows_v.at[b],
            out.at[pl.ds((wid * NCHUNK + c) * CHUNK, CHUNK)],
            sem_out.at[b],
        )

    def wait_out(c, b):
        pltpu.make_async_copy(
            rows_v.at[b],
            out.at[pl.ds((wid * NCHUNK + c) * CHUNK, CHUNK)],
            sem_out.at[b],
        ).wait()

    pre(0)
    wait_pre(0)
    fire_gathers(0, 0)
    pre(1)
    pltpu.make_async_copy(
        idx_hbm.at[wid, pl.ds(2, NCHUNK - 2)], idx_v.at[pl.ds(2, NCHUNK - 2)], sem_idx
    ).wait()

    def pair_body(t, carry):
        c0 = 2 * t
        c1 = 2 * t + 1
        wait_gathers(c0, 0)
        start_out(c0, 0)
        wait_pre(1)
        fire_gathers(c1, 1)
        wait_out(c0, 0)
        pre(0)
        wait_gathers(c1, 1)
        start_out(c1, 1)
        wait_pre(0)
        fire_gathers(c0 + 2, 0)
        wait_out(c1, 1)
        pre(1)
        return carry

    lax.fori_loop(0, NT - 1, pair_body, 0)

    c0 = NCHUNK - 2
    c1 = NCHUNK - 1
    wait_gathers(c0, 0)
    start_out(c0, 0)
    wait_pre(1)
    fire_gathers(c1, 1)
    wait_out(c0, 0)
    wait_gathers(c1, 1)
    start_out(c1, 1)
    wait_out(c1, 1)


@functools.partial(jax.jit, donate_argnums=())
def _emb(table, idx4, pos_tmpl):
    mesh = plsc.VectorSubcoreMesh(core_axis_name="c", subcore_axis_name="s")
    run = pl.kernel(
        _body,
        mesh=mesh,
        compiler_params=pltpu.CompilerParams(use_tc_tiling_on_sc=False),
        out_type=jax.ShapeDtypeStruct((ROWS, EMBED_DIM), jnp.float32),
        scratch_types=[
            pltpu.VMEM((NCHUNK, NSUB, SUB), jnp.int32),
            pltpu.VMEM((2, CHUNK, EMBED_DIM), jnp.float32),
            pltpu.VMEM_SHARED((CHUNK, EMBED_DIM), jnp.float32),
            pltpu.SemaphoreType.DMA((2,)),
            pltpu.SemaphoreType.DMA((2,)),
            pltpu.SemaphoreType.DMA((2,)),
            pltpu.SemaphoreType.DMA,
        ],
    )
    return run(table, idx4, pos_tmpl)


def kernel(token_emb, x):
    idx4 = x.astype(jnp.int32).reshape(NW, NCHUNK, NSUB, SUB)
    out = _emb(token_emb, idx4, _POS_TMPL)
    return out.reshape(BATCH, SEQ_LEN, EMBED_DIM)
